# baseline (speedup 1.0000x reference)
import functools

import jax
import jax.numpy as jnp
from jax import lax
from jax.experimental import pallas as pl
from jax.experimental.pallas import tpu as pltpu
from jax.experimental.pallas import tpu_sc as plsc

E = 8
K = 2
D = 1024
F = 1408
T = 2048

TM = 256
NT = 24
NP = NT * TM

NC = 2
NS = 16
NW = NC * NS
TPW = T // NW
NV = TPW // 16

_MESH = plsc.VectorSubcoreMesh(core_axis_name="c", subcore_axis_name="s")


def _wid():
    return lax.axis_index("s") * NC + lax.axis_index("c")


def _router_body(x_ref, wr_ref, out_ref):
    prod = lax.dot_general(
        wr_ref[...], x_ref[...],
        (((1,), (1,)), ((), ())),
        preferred_element_type=jnp.float32,
    )
    out_ref[...] = prod[:E, :]


def _router_logits(hidden, wr_pad):
    return pl.pallas_call(
        _router_body,
        out_shape=jax.ShapeDtypeStruct((E, T), jnp.float32),
    )(hidden, wr_pad)


def _route_body(logits_hbm, e_hbm, w_hbm, r_hbm, c_hbm,
                lbuf, ebuf, wbuf, rbuf, cnt_ref):
    wid = _wid()
    base = wid * TPW
    for e in range(E):
        pltpu.sync_copy(logits_hbm.at[e, pl.ds(base, TPW)], lbuf.at[e])
    cnt_ref[...] = jnp.zeros((16,), jnp.int32)
    lane = lax.iota(jnp.int32, 16)
    for v in range(NV):
        sl = pl.ds(v * 16, 16)
        m1 = lbuf[0, sl]
        e1 = jnp.zeros((16,), jnp.int32)
        m2 = jnp.full((16,), -jnp.inf, jnp.float32)
        e2 = jnp.zeros((16,), jnp.int32)
        for e in range(1, E):
            l = lbuf[e, sl]
            gt1 = l > m1
            gt2 = l > m2
            es = jnp.full((16,), e, jnp.int32)
            m2n = jnp.where(gt1, m1, jnp.where(gt2, l, m2))
            e2n = jnp.where(gt1, e1, jnp.where(gt2, es, e2))
            m1 = jnp.where(gt1, l, m1)
            e1 = jnp.where(gt1, es, e1)
            m2 = m2n
            e2 = e2n
        w0 = 1.0 / (1.0 + jnp.exp(m2 - m1))
        w1 = 1.0 - w0
        for slot, ev, wv in ((0, e1, w0), (1, e2, w1)):
            pre = plsc.load_gather(cnt_ref, [ev])
            dup = jnp.zeros((16,), jnp.int32)
            incr = jnp.zeros((16,), jnp.int32)
            for e in range(E):
                mask = ev == e
                mi = mask.astype(jnp.int32)
                cs = plsc.cumsum(mi)
                dup = dup + jnp.where(mask, cs - mi, 0)
                tot = jnp.sum(mi)
                incr = jnp.where(lane == e, incr + tot, incr)
            cnt_ref[...] = cnt_ref[...] + incr
            ebuf[slot, sl] = ev
            wbuf[slot, sl] = wv
            rbuf[slot, sl] = pre + dup
    for slot in range(K):
        pltpu.sync_copy(ebuf.at[slot], e_hbm.at[slot, pl.ds(base, TPW)])
        pltpu.sync_copy(wbuf.at[slot], w_hbm.at[slot, pl.ds(base, TPW)])
        pltpu.sync_copy(rbuf.at[slot], r_hbm.at[slot, pl.ds(base, TPW)])
    pltpu.sync_copy(cnt_ref, c_hbm.at[wid])


_route = pl.kernel(
    _route_body,
    out_type=(
        jax.ShapeDtypeStruct((K, T), jnp.int32),
        jax.ShapeDtypeStruct((K, T), jnp.float32),
        jax.ShapeDtypeStruct((K, T), jnp.int32),
        jax.ShapeDtypeStruct((NW, 16), jnp.int32),
    ),
    mesh=_MESH,
    compiler_params=pltpu.CompilerParams(needs_layout_passes=False),
    scratch_types=[
        pltpu.VMEM((E, TPW), jnp.float32),
        pltpu.VMEM((K, TPW), jnp.int32),
        pltpu.VMEM((K, TPW), jnp.float32),
        pltpu.VMEM((K, TPW), jnp.int32),
        pltpu.VMEM((16,), jnp.int32),
    ],
)


def _dispatch_body(c_hbm, e_hbm, w_hbm, r_hbm, x_hbm,
                   xs_hbm, ws_hbm, inv_hbm, meta_hbm,
                   cbuf, ebuf, wbuf, rbuf, base_ref, posb, pos_all,
                   wrows, invb, xbuf, mrow, sem):
    wid = _wid()
    base = wid * TPW
    pltpu.sync_copy(c_hbm, cbuf)
    lane = lax.iota(jnp.int32, 16)
    tot = jnp.zeros((16,), jnp.int32)
    pre = jnp.zeros((16,), jnp.int32)
    for t in range(NW):
        row = cbuf[t]
        tot = tot + row
        pre = pre + jnp.where(t < wid, row, 0)
    padded = (tot + (TM - 1)) & (-TM)
    inc = plsc.cumsum(padded)
    gstart = inc - padded
    base_ref[...] = gstart + pre
    for slot in range(K):
        pltpu.sync_copy(e_hbm.at[slot, pl.ds(base, TPW)], ebuf.at[slot])
        pltpu.sync_copy(w_hbm.at[slot, pl.ds(base, TPW)], wbuf.at[slot])
        pltpu.sync_copy(r_hbm.at[slot, pl.ds(base, TPW)], rbuf.at[slot])
    for v in range(NV):
        sl = pl.ds(v * 16, 16)
        for slot in range(K):
            ev = ebuf[slot, sl]
            pos = plsc.load_gather(base_ref, [ev]) + rbuf[slot, sl]
            posb[slot, sl] = pos
            pos_all[pl.ds(slot * TPW + v * 16, 16)] = pos
            plsc.store_scatter(invb, [(v * 16 + lane) * K + slot], pos)
            plsc.store_scatter(
                wrows,
                [slot * TPW + v * 16 + lane, jnp.zeros((16,), jnp.int32)],
                wbuf[slot, sl])
    pltpu.sync_copy(x_hbm.at[pl.ds(base, TPW)], xbuf)
    pltpu.async_copy(xbuf, xs_hbm.at[posb.at[0]], sem).wait()
    pltpu.async_copy(xbuf, xs_hbm.at[posb.at[1]], sem).wait()
    pltpu.async_copy(wrows, ws_hbm.at[pos_all], sem).wait()
    pltpu.sync_copy(invb, inv_hbm.at[pl.ds(wid * K * TPW, K * TPW)])
    pend = gstart + padded
    active8 = lane < E
    tpos = wid * TM
    te = jnp.sum(jnp.where(active8, (tpos >= pend).astype(jnp.int32), 0))
    ptotal = jnp.sum(jnp.where(active8, padded, 0))
    tv = (tpos < ptotal).astype(jnp.int32)
    te_c = jnp.minimum(te, E - 1)
    mrow[...] = jnp.where(lane == 0, te_c, jnp.where(lane == 1, tv, 0))
    pltpu.sync_copy(mrow, meta_hbm.at[wid])


_dispatch = pl.kernel(
    _dispatch_body,
    out_type=(
        jax.ShapeDtypeStruct((NP, D), jnp.float32),
        jax.ShapeDtypeStruct((NP, 128), jnp.float32),
        jax.ShapeDtypeStruct((K * T,), jnp.int32),
        jax.ShapeDtypeStruct((NW, 16), jnp.int32),
    ),
    mesh=_MESH,
    compiler_params=pltpu.CompilerParams(needs_layout_passes=False),
    scratch_types=[
        pltpu.VMEM((NW, 16), jnp.int32),
        pltpu.VMEM((K, TPW), jnp.int32),
        pltpu.VMEM((K, TPW), jnp.float32),
        pltpu.VMEM((K, TPW), jnp.int32),
        pltpu.VMEM((16,), jnp.int32),
        pltpu.VMEM((K, TPW), jnp.int32),
        pltpu.VMEM((K * TPW,), jnp.int32),
        pltpu.VMEM((K * TPW, 128), jnp.float32),
        pltpu.VMEM((K * TPW,), jnp.int32),
        pltpu.VMEM((TPW, D), jnp.float32),
        pltpu.VMEM((16,), jnp.int32),
        pltpu.SemaphoreType.DMA,
    ],
)


def _mlp_body(te_ref, tv_ref, x_ref, gup_ref, down_ref, w_ref, y_ref):
    @pl.when(tv_ref[pl.program_id(0)] == 1)
    def _():
        x = x_ref[...].astype(jnp.bfloat16)
        gu = jnp.dot(x, gup_ref[0], preferred_element_type=jnp.float32)
        gate = gu[:, :F]
        up = gu[:, F:]
        h = (gate * jax.nn.sigmoid(gate) * up * w_ref[:, :1]).astype(jnp.bfloat16)
        y_ref[...] = jnp.dot(h, down_ref[0], preferred_element_type=jnp.float32)


def _grouped_mlp(tile_expert, tile_valid, x_sorted, gup, down, w_sorted):
    grid_spec = pltpu.PrefetchScalarGridSpec(
        num_scalar_prefetch=2,
        grid=(NT,),
        in_specs=[
            pl.BlockSpec((TM, D), lambda i, te, tv: (i, 0)),
            pl.BlockSpec((1, D, 2 * F), lambda i, te, tv: (te[i], 0, 0)),
            pl.BlockSpec((1, F, D), lambda i, te, tv: (te[i], 0, 0)),
            pl.BlockSpec((TM, 128), lambda i, te, tv: (i, 0)),
        ],
        out_specs=pl.BlockSpec((TM, D), lambda i, te, tv: (i, 0)),
    )
    return pl.pallas_call(
        _mlp_body,
        grid_spec=grid_spec,
        out_shape=jax.ShapeDtypeStruct((NP, D), jnp.float32),
    )(tile_expert, tile_valid, x_sorted, gup, down, w_sorted)


_CHUNK = 16
_NCH = TPW // _CHUNK


def _combine_body(inv_hbm, y_hbm, out_hbm, ib, yb, ob, sem):
    wid = _wid()
    for c in range(_NCH):
        pltpu.sync_copy(
            inv_hbm.at[pl.ds(wid * K * TPW + c * K * _CHUNK, K * _CHUNK)],
            ib.at[c])
    for c in range(_NCH):
        pltpu.async_copy(y_hbm.at[ib.at[c]], yb, sem).wait()

        def body(j, carry):
            for col in range(D // 16):
                s = pl.ds(col * 16, 16)
                ob[j, s] = yb[2 * j, s] + yb[2 * j + 1, s]
            return carry

        lax.fori_loop(0, _CHUNK, body, 0)
        pltpu.sync_copy(
            ob, out_hbm.at[pl.ds(wid * TPW + c * _CHUNK, _CHUNK)])


_combine = pl.kernel(
    _combine_body,
    out_type=jax.ShapeDtypeStruct((T, D), jnp.float32),
    mesh=_MESH,
    compiler_params=pltpu.CompilerParams(needs_layout_passes=False),
    scratch_types=[
        pltpu.VMEM((_NCH, K * _CHUNK), jnp.int32),
        pltpu.VMEM((K * _CHUNK, D), jnp.float32),
        pltpu.VMEM((_CHUNK, D), jnp.float32),
        pltpu.SemaphoreType.DMA,
    ],
)


def kernel(hidden_states, router_weight, gate_up_weight, down_weight):
    wr_pad = jnp.zeros((128, D), jnp.float32).at[:E, :].set(router_weight.T)
    logits_t = _router_logits(hidden_states, wr_pad)
    eidx, wts, ranks, counts = _route(logits_t)
    x_sorted, w_sorted, inv, meta = _dispatch(
        counts, eidx, wts, ranks, hidden_states)
    tile_expert = meta[:NT, 0]
    tile_valid = meta[:NT, 1]
    y = _grouped_mlp(tile_expert, tile_valid, x_sorted,
                     gate_up_weight.astype(jnp.bfloat16),
                     down_weight.astype(jnp.bfloat16), w_sorted)
    return _combine(inv, y)


# speedup vs baseline: 1.2467x; 1.2467x over previous
import functools

import jax
import jax.numpy as jnp
from jax import lax
from jax.experimental import pallas as pl
from jax.experimental.pallas import tpu as pltpu
from jax.experimental.pallas import tpu_sc as plsc

E = 8
K = 2
D = 1024
F = 1408
T = 2048

TM = 256
NT = 24
NP = NT * TM

NC = 2
NS = 16
NW = NC * NS
TPW = T // NW
NV = TPW // 16

_MESH = plsc.VectorSubcoreMesh(core_axis_name="c", subcore_axis_name="s")


def _wid():
    return lax.axis_index("s") * NC + lax.axis_index("c")


def _router_body(x_ref, wr_ref, out_ref):
    prod = lax.dot_general(
        wr_ref[...], x_ref[...],
        (((1,), (1,)), ((), ())),
        preferred_element_type=jnp.float32,
    )
    out_ref[...] = prod[:E, :]


def _router_logits(hidden, wr_pad):
    return pl.pallas_call(
        _router_body,
        out_shape=jax.ShapeDtypeStruct((E, T), jnp.float32),
    )(hidden, wr_pad)


def _route_body(logits_hbm, e_hbm, w_hbm, r_hbm, c_hbm,
                lbuf, ebuf, wbuf, rbuf, cnt_ref):
    wid = _wid()
    base = wid * TPW
    for e in range(E):
        pltpu.sync_copy(logits_hbm.at[e, pl.ds(base, TPW)], lbuf.at[e])
    cnt_ref[...] = jnp.zeros((16,), jnp.int32)
    lane = lax.iota(jnp.int32, 16)
    for v in range(NV):
        sl = pl.ds(v * 16, 16)
        m1 = lbuf[0, sl]
        e1 = jnp.zeros((16,), jnp.int32)
        m2 = jnp.full((16,), -jnp.inf, jnp.float32)
        e2 = jnp.zeros((16,), jnp.int32)
        for e in range(1, E):
            l = lbuf[e, sl]
            gt1 = l > m1
            gt2 = l > m2
            es = jnp.full((16,), e, jnp.int32)
            m2n = jnp.where(gt1, m1, jnp.where(gt2, l, m2))
            e2n = jnp.where(gt1, e1, jnp.where(gt2, es, e2))
            m1 = jnp.where(gt1, l, m1)
            e1 = jnp.where(gt1, es, e1)
            m2 = m2n
            e2 = e2n
        w0 = 1.0 / (1.0 + jnp.exp(m2 - m1))
        w1 = 1.0 - w0
        for slot, ev, wv in ((0, e1, w0), (1, e2, w1)):
            pre = plsc.load_gather(cnt_ref, [ev])
            dup = jnp.zeros((16,), jnp.int32)
            incr = jnp.zeros((16,), jnp.int32)
            for e in range(E):
                mask = ev == e
                mi = mask.astype(jnp.int32)
                cs = plsc.cumsum(mi)
                dup = dup + jnp.where(mask, cs - mi, 0)
                tot = jnp.sum(mi)
                incr = jnp.where(lane == e, incr + tot, incr)
            cnt_ref[...] = cnt_ref[...] + incr
            ebuf[slot, sl] = ev
            wbuf[slot, sl] = wv
            rbuf[slot, sl] = pre + dup
    for slot in range(K):
        pltpu.sync_copy(ebuf.at[slot], e_hbm.at[slot, pl.ds(base, TPW)])
        pltpu.sync_copy(wbuf.at[slot], w_hbm.at[slot, pl.ds(base, TPW)])
        pltpu.sync_copy(rbuf.at[slot], r_hbm.at[slot, pl.ds(base, TPW)])
    pltpu.sync_copy(cnt_ref, c_hbm.at[wid])


_route = pl.kernel(
    _route_body,
    out_type=(
        jax.ShapeDtypeStruct((K, T), jnp.int32),
        jax.ShapeDtypeStruct((K, T), jnp.float32),
        jax.ShapeDtypeStruct((K, T), jnp.int32),
        jax.ShapeDtypeStruct((NW, 16), jnp.int32),
    ),
    mesh=_MESH,
    compiler_params=pltpu.CompilerParams(needs_layout_passes=False),
    scratch_types=[
        pltpu.VMEM((E, TPW), jnp.float32),
        pltpu.VMEM((K, TPW), jnp.int32),
        pltpu.VMEM((K, TPW), jnp.float32),
        pltpu.VMEM((K, TPW), jnp.int32),
        pltpu.VMEM((16,), jnp.int32),
    ],
)


def _dispatch_body(c_hbm, e_hbm, w_hbm, r_hbm, x_hbm,
                   xs_hbm, ws_hbm, inv_hbm, meta_hbm,
                   cbuf, ebuf, wbuf, rbuf, base_ref, posb, pos_all,
                   wrows, invb, xbuf, mrow, sem):
    wid = _wid()
    base = wid * TPW
    pltpu.sync_copy(c_hbm, cbuf)
    lane = lax.iota(jnp.int32, 16)
    tot = jnp.zeros((16,), jnp.int32)
    pre = jnp.zeros((16,), jnp.int32)
    for t in range(NW):
        row = cbuf[t]
        tot = tot + row
        pre = pre + jnp.where(t < wid, row, 0)
    padded = (tot + (TM - 1)) & (-TM)
    inc = plsc.cumsum(padded)
    gstart = inc - padded
    base_ref[...] = gstart + pre
    for slot in range(K):
        pltpu.sync_copy(e_hbm.at[slot, pl.ds(base, TPW)], ebuf.at[slot])
        pltpu.sync_copy(w_hbm.at[slot, pl.ds(base, TPW)], wbuf.at[slot])
        pltpu.sync_copy(r_hbm.at[slot, pl.ds(base, TPW)], rbuf.at[slot])
    for v in range(NV):
        sl = pl.ds(v * 16, 16)
        for slot in range(K):
            ev = ebuf[slot, sl]
            pos = plsc.load_gather(base_ref, [ev]) + rbuf[slot, sl]
            posb[slot, sl] = pos
            pos_all[pl.ds(slot * TPW + v * 16, 16)] = pos
            plsc.store_scatter(invb, [(v * 16 + lane) * K + slot], pos)
            plsc.store_scatter(
                wrows,
                [slot * TPW + v * 16 + lane, jnp.zeros((16,), jnp.int32)],
                wbuf[slot, sl])
    pltpu.sync_copy(x_hbm.at[pl.ds(base, TPW)], xbuf)
    cp0 = pltpu.async_copy(xbuf, xs_hbm.at[posb.at[0]], sem)
    cp1 = pltpu.async_copy(xbuf, xs_hbm.at[posb.at[1]], sem)
    cp2 = pltpu.async_copy(wrows, ws_hbm.at[pos_all], sem)
    cp0.wait()
    cp1.wait()
    cp2.wait()
    pltpu.sync_copy(invb, inv_hbm.at[pl.ds(wid * K * TPW, K * TPW)])
    pend = gstart + padded
    active8 = lane < E
    tpos = wid * TM
    te = jnp.sum(jnp.where(active8, (tpos >= pend).astype(jnp.int32), 0))
    ptotal = jnp.sum(jnp.where(active8, padded, 0))
    tv = (tpos < ptotal).astype(jnp.int32)
    te_c = jnp.minimum(te, E - 1)
    mrow[...] = jnp.where(lane == 0, te_c, jnp.where(lane == 1, tv, 0))
    pltpu.sync_copy(mrow, meta_hbm.at[wid])


_dispatch = pl.kernel(
    _dispatch_body,
    out_type=(
        jax.ShapeDtypeStruct((NP, D), jnp.float32),
        jax.ShapeDtypeStruct((NP, 128), jnp.float32),
        jax.ShapeDtypeStruct((K * T,), jnp.int32),
        jax.ShapeDtypeStruct((NW, 16), jnp.int32),
    ),
    mesh=_MESH,
    compiler_params=pltpu.CompilerParams(needs_layout_passes=False),
    scratch_types=[
        pltpu.VMEM((NW, 16), jnp.int32),
        pltpu.VMEM((K, TPW), jnp.int32),
        pltpu.VMEM((K, TPW), jnp.float32),
        pltpu.VMEM((K, TPW), jnp.int32),
        pltpu.VMEM((16,), jnp.int32),
        pltpu.VMEM((K, TPW), jnp.int32),
        pltpu.VMEM((K * TPW,), jnp.int32),
        pltpu.VMEM((K * TPW, 128), jnp.float32),
        pltpu.VMEM((K * TPW,), jnp.int32),
        pltpu.VMEM((TPW, D), jnp.float32),
        pltpu.VMEM((16,), jnp.int32),
        pltpu.SemaphoreType.DMA,
    ],
)


def _mlp_body(te_ref, tv_ref, x_ref, gup_ref, down_ref, w_ref, y_ref):
    @pl.when(tv_ref[pl.program_id(0)] == 1)
    def _():
        x = x_ref[...]
        gu = jnp.dot(x, gup_ref[0], preferred_element_type=jnp.float32)
        gate = gu[:, :F]
        up = gu[:, F:]
        h = gate * jax.nn.sigmoid(gate) * up * w_ref[:, :1]
        y_ref[...] = jnp.dot(h, down_ref[0], preferred_element_type=jnp.float32)


def _grouped_mlp(tile_expert, tile_valid, x_sorted, gup, down, w_sorted):
    grid_spec = pltpu.PrefetchScalarGridSpec(
        num_scalar_prefetch=2,
        grid=(NT,),
        in_specs=[
            pl.BlockSpec((TM, D), lambda i, te, tv: (i, 0)),
            pl.BlockSpec((1, D, 2 * F), lambda i, te, tv: (te[i], 0, 0)),
            pl.BlockSpec((1, F, D), lambda i, te, tv: (te[i], 0, 0)),
            pl.BlockSpec((TM, 128), lambda i, te, tv: (i, 0)),
        ],
        out_specs=pl.BlockSpec((TM, D), lambda i, te, tv: (i, 0)),
    )
    return pl.pallas_call(
        _mlp_body,
        grid_spec=grid_spec,
        out_shape=jax.ShapeDtypeStruct((NP, D), jnp.float32),
    )(tile_expert, tile_valid, x_sorted, gup, down, w_sorted)


_CHUNK = 16
_NCH = TPW // _CHUNK


def _combine_body(inv_hbm, y_hbm, out_hbm, ib, yb0, yb1, ob0, ob1, sem0, sem1, osem):
    wid = _wid()
    for c in range(_NCH):
        pltpu.sync_copy(
            inv_hbm.at[pl.ds(wid * K * TPW + c * K * _CHUNK, K * _CHUNK)],
            ib.at[c])
    ybs = (yb0, yb1)
    obs = (ob0, ob1)
    sems = (sem0, sem1)
    cps = [None, None]
    cps[0] = pltpu.async_copy(y_hbm.at[ib.at[0]], yb0, sem0)
    for c in range(_NCH):
        p = c % 2
        if c + 1 < _NCH:
            cps[1 - p] = pltpu.async_copy(
                y_hbm.at[ib.at[c + 1]], ybs[1 - p], sem=sems[1 - p])
        cps[p].wait()
        yb = ybs[p]
        ob = obs[p]
        if c >= 2:
            cps_out[p].wait()

        def body(j, carry):
            for col in range(D // 16):
                s = pl.ds(col * 16, 16)
                ob[j, s] = yb[2 * j, s] + yb[2 * j + 1, s]
            return carry

        lax.fori_loop(0, _CHUNK, body, 0)
        cp_o = pltpu.async_copy(
            ob, out_hbm.at[pl.ds(wid * TPW + c * _CHUNK, _CHUNK)], osem)
        if c == 0:
            cps_out = [cp_o, None]
        else:
            cps_out[p] = cp_o
    cps_out[0].wait()
    cps_out[1].wait()


_combine = pl.kernel(
    _combine_body,
    out_type=jax.ShapeDtypeStruct((T, D), jnp.float32),
    mesh=_MESH,
    compiler_params=pltpu.CompilerParams(needs_layout_passes=False),
    scratch_types=[
        pltpu.VMEM((_NCH, K * _CHUNK), jnp.int32),
        pltpu.VMEM((K * _CHUNK, D), jnp.float32),
        pltpu.VMEM((K * _CHUNK, D), jnp.float32),
        pltpu.VMEM((_CHUNK, D), jnp.float32),
        pltpu.VMEM((_CHUNK, D), jnp.float32),
        pltpu.SemaphoreType.DMA,
        pltpu.SemaphoreType.DMA,
        pltpu.SemaphoreType.DMA,
    ],
)


def kernel(hidden_states, router_weight, gate_up_weight, down_weight):
    wr_pad = jnp.zeros((128, D), jnp.float32).at[:E, :].set(router_weight.T)
    logits_t = _router_logits(hidden_states, wr_pad)
    eidx, wts, ranks, counts = _route(logits_t)
    x_sorted, w_sorted, inv, meta = _dispatch(
        counts, eidx, wts, ranks, hidden_states)
    tile_expert = meta[:NT, 0]
    tile_valid = meta[:NT, 1]
    y = _grouped_mlp(tile_expert, tile_valid, x_sorted,
                     gate_up_weight, down_weight, w_sorted)
    return _combine(inv, y)


# speedup vs baseline: 1.3511x; 1.0838x over previous
import functools

import jax
import jax.numpy as jnp
from jax import lax
from jax.experimental import pallas as pl
from jax.experimental.pallas import tpu as pltpu
from jax.experimental.pallas import tpu_sc as plsc

E = 8
K = 2
D = 1024
F = 1408
T = 2048

TM = 256
NT = 24
NP = NT * TM

NC = 2
NS = 16
NW = NC * NS
TPW = T // NW
NV = TPW // 16

_MESH = plsc.VectorSubcoreMesh(core_axis_name="c", subcore_axis_name="s")


def _wid():
    return lax.axis_index("s") * NC + lax.axis_index("c")


def _router_body(x_ref, wr_ref, out_ref):
    prod = lax.dot_general(
        wr_ref[...], x_ref[...],
        (((1,), (1,)), ((), ())),
        preferred_element_type=jnp.float32,
    )
    out_ref[...] = prod[:E, :]


def _router_logits(hidden, wr_pad):
    return pl.pallas_call(
        _router_body,
        out_shape=jax.ShapeDtypeStruct((E, T), jnp.float32),
    )(hidden, wr_pad)


def _route_body(logits_hbm, e_hbm, w_hbm, r_hbm, c_hbm,
                lbuf, ebuf, wbuf, rbuf, cnt_ref):
    wid = _wid()
    base = wid * TPW
    for e in range(E):
        pltpu.sync_copy(logits_hbm.at[e, pl.ds(base, TPW)], lbuf.at[e])
    cnt_ref[...] = jnp.zeros((16,), jnp.int32)
    lane = lax.iota(jnp.int32, 16)
    for v in range(NV):
        sl = pl.ds(v * 16, 16)
        m1 = lbuf[0, sl]
        e1 = jnp.zeros((16,), jnp.int32)
        m2 = jnp.full((16,), -jnp.inf, jnp.float32)
        e2 = jnp.zeros((16,), jnp.int32)
        for e in range(1, E):
            l = lbuf[e, sl]
            gt1 = l > m1
            gt2 = l > m2
            es = jnp.full((16,), e, jnp.int32)
            m2n = jnp.where(gt1, m1, jnp.where(gt2, l, m2))
            e2n = jnp.where(gt1, e1, jnp.where(gt2, es, e2))
            m1 = jnp.where(gt1, l, m1)
            e1 = jnp.where(gt1, es, e1)
            m2 = m2n
            e2 = e2n
        w0 = 1.0 / (1.0 + jnp.exp(m2 - m1))
        w1 = 1.0 - w0
        for slot, ev, wv in ((0, e1, w0), (1, e2, w1)):
            pre = plsc.load_gather(cnt_ref, [ev])
            dup = jnp.zeros((16,), jnp.int32)
            incr = jnp.zeros((16,), jnp.int32)
            for e in range(E):
                mask = ev == e
                mi = mask.astype(jnp.int32)
                cs = plsc.cumsum(mi)
                dup = dup + jnp.where(mask, cs - mi, 0)
                tot = jnp.sum(mi)
                incr = jnp.where(lane == e, incr + tot, incr)
            cnt_ref[...] = cnt_ref[...] + incr
            ebuf[slot, sl] = ev
            wbuf[slot, sl] = wv
            rbuf[slot, sl] = pre + dup
    for slot in range(K):
        pltpu.sync_copy(ebuf.at[slot], e_hbm.at[slot, pl.ds(base, TPW)])
        pltpu.sync_copy(wbuf.at[slot], w_hbm.at[slot, pl.ds(base, TPW)])
        pltpu.sync_copy(rbuf.at[slot], r_hbm.at[slot, pl.ds(base, TPW)])
    pltpu.sync_copy(cnt_ref, c_hbm.at[wid])


_route = pl.kernel(
    _route_body,
    out_type=(
        jax.ShapeDtypeStruct((K, T), jnp.int32),
        jax.ShapeDtypeStruct((K, T), jnp.float32),
        jax.ShapeDtypeStruct((K, T), jnp.int32),
        jax.ShapeDtypeStruct((NW, 16), jnp.int32),
    ),
    mesh=_MESH,
    compiler_params=pltpu.CompilerParams(needs_layout_passes=False),
    scratch_types=[
        pltpu.VMEM((E, TPW), jnp.float32),
        pltpu.VMEM((K, TPW), jnp.int32),
        pltpu.VMEM((K, TPW), jnp.float32),
        pltpu.VMEM((K, TPW), jnp.int32),
        pltpu.VMEM((16,), jnp.int32),
    ],
)


def _dispatch_body(c_hbm, e_hbm, w_hbm, r_hbm, x_hbm,
                   xs_hbm, ws_hbm, inv_hbm, meta_hbm,
                   cbuf, ebuf, wbuf, rbuf, base_ref, posb, pos_all,
                   wrows, invb, xbuf, mrow, sem):
    wid = _wid()
    base = wid * TPW
    pltpu.sync_copy(c_hbm, cbuf)
    lane = lax.iota(jnp.int32, 16)
    tot = jnp.zeros((16,), jnp.int32)
    pre = jnp.zeros((16,), jnp.int32)
    for t in range(NW):
        row = cbuf[t]
        tot = tot + row
        pre = pre + jnp.where(t < wid, row, 0)
    padded = (tot + (TM - 1)) & (-TM)
    inc = plsc.cumsum(padded)
    gstart = inc - padded
    base_ref[...] = gstart + pre
    for slot in range(K):
        pltpu.sync_copy(e_hbm.at[slot, pl.ds(base, TPW)], ebuf.at[slot])
        pltpu.sync_copy(w_hbm.at[slot, pl.ds(base, TPW)], wbuf.at[slot])
        pltpu.sync_copy(r_hbm.at[slot, pl.ds(base, TPW)], rbuf.at[slot])
    for v in range(NV):
        sl = pl.ds(v * 16, 16)
        for slot in range(K):
            ev = ebuf[slot, sl]
            pos = plsc.load_gather(base_ref, [ev]) + rbuf[slot, sl]
            posb[slot, sl] = pos
            pos_all[pl.ds(slot * TPW + v * 16, 16)] = pos
            plsc.store_scatter(invb, [(v * 16 + lane) * K + slot], pos)
            plsc.store_scatter(
                wrows,
                [slot * TPW + v * 16 + lane, jnp.zeros((16,), jnp.int32)],
                wbuf[slot, sl])
    pltpu.sync_copy(x_hbm.at[pl.ds(base, TPW)], xbuf)
    cp0 = pltpu.async_copy(xbuf, xs_hbm.at[posb.at[0]], sem)
    cp1 = pltpu.async_copy(xbuf, xs_hbm.at[posb.at[1]], sem)
    cp2 = pltpu.async_copy(wrows, ws_hbm.at[pos_all], sem)
    cp0.wait()
    cp1.wait()
    cp2.wait()
    pltpu.sync_copy(invb, inv_hbm.at[pl.ds(wid * K * TPW, K * TPW)])
    pend = gstart + padded
    active8 = lane < E
    tpos = wid * TM
    te = jnp.sum(jnp.where(active8, (tpos >= pend).astype(jnp.int32), 0))
    ptotal = jnp.sum(jnp.where(active8, padded, 0))
    tv = (tpos < ptotal).astype(jnp.int32)
    te_c = jnp.minimum(te, E - 1)
    mrow[...] = jnp.where(lane == 0, te_c, jnp.where(lane == 1, tv, 0))
    pltpu.sync_copy(mrow, meta_hbm.at[wid])


_dispatch = pl.kernel(
    _dispatch_body,
    out_type=(
        jax.ShapeDtypeStruct((NP, D), jnp.float32),
        jax.ShapeDtypeStruct((NP, 128), jnp.float32),
        jax.ShapeDtypeStruct((K * T,), jnp.int32),
        jax.ShapeDtypeStruct((NW, 16), jnp.int32),
    ),
    mesh=_MESH,
    compiler_params=pltpu.CompilerParams(needs_layout_passes=False),
    scratch_types=[
        pltpu.VMEM((NW, 16), jnp.int32),
        pltpu.VMEM((K, TPW), jnp.int32),
        pltpu.VMEM((K, TPW), jnp.float32),
        pltpu.VMEM((K, TPW), jnp.int32),
        pltpu.VMEM((16,), jnp.int32),
        pltpu.VMEM((K, TPW), jnp.int32),
        pltpu.VMEM((K * TPW,), jnp.int32),
        pltpu.VMEM((K * TPW, 128), jnp.float32),
        pltpu.VMEM((K * TPW,), jnp.int32),
        pltpu.VMEM((TPW, D), jnp.float32),
        pltpu.VMEM((16,), jnp.int32),
        pltpu.SemaphoreType.DMA,
    ],
)


def _mlp_body(te_ref, tv_ref, first_ref, slot_ref, nval_ref, nexp_ref,
              x_ref, gup_hbm, down_hbm, w_ref, y_ref,
              gup_buf, down_buf, sems):
    i = pl.program_id(0)

    def start_copy(e, s):
        pltpu.make_async_copy(gup_hbm.at[e], gup_buf.at[s], sems.at[s]).start()
        pltpu.make_async_copy(down_hbm.at[e], down_buf.at[s], sems.at[s]).start()

    @pl.when(i == 0)
    def _():
        start_copy(te_ref[0], 0)

    @pl.when((first_ref[i] == 1) & (nval_ref[i] == 1))
    def _():
        start_copy(nexp_ref[i], 1 - slot_ref[i])

    @pl.when((first_ref[i] == 1) & (tv_ref[i] == 1))
    def _():
        s = slot_ref[i]
        pltpu.make_async_copy(gup_hbm.at[0], gup_buf.at[s], sems.at[s]).wait()
        pltpu.make_async_copy(down_hbm.at[0], down_buf.at[s], sems.at[s]).wait()

    @pl.when(tv_ref[i] == 1)
    def _():
        s = slot_ref[i]
        x = x_ref[...]
        gu = jnp.dot(x, gup_buf[s], preferred_element_type=jnp.float32)
        gate = gu[:, :F]
        up = gu[:, F:]
        h = gate * jax.nn.sigmoid(gate) * up * w_ref[:, :1]
        y_ref[...] = jnp.dot(h, down_buf[s], preferred_element_type=jnp.float32)


def _grouped_mlp(tile_expert, tile_valid, x_sorted, gup, down, w_sorted):
    te = tile_expert
    tv = tile_valid
    prev_te = jnp.concatenate([jnp.full((1,), -1, jnp.int32), te[:-1]])
    first = ((te != prev_te) & (tv == 1)).astype(jnp.int32)
    ordinal = jnp.cumsum(first).astype(jnp.int32) - first
    slot = ordinal % 2
    idx = jnp.arange(NT, dtype=jnp.int32)
    later_first = (first == 1)
    nxt = jnp.min(jnp.where(later_first & (idx[None, :] > idx[:, None]),
                            idx[None, :], NT), axis=1).astype(jnp.int32)
    nval = ((nxt < NT) & (first == 1)).astype(jnp.int32)
    nexp = te[jnp.minimum(nxt, NT - 1)]

    grid_spec = pltpu.PrefetchScalarGridSpec(
        num_scalar_prefetch=6,
        grid=(NT,),
        in_specs=[
            pl.BlockSpec((TM, D), lambda i, *_: (i, 0)),
            pl.BlockSpec(memory_space=pl.ANY),
            pl.BlockSpec(memory_space=pl.ANY),
            pl.BlockSpec((TM, 128), lambda i, *_: (i, 0)),
        ],
        out_specs=pl.BlockSpec((TM, D), lambda i, *_: (i, 0)),
        scratch_shapes=[
            pltpu.VMEM((2, D, 2 * F), jnp.float32),
            pltpu.VMEM((2, F, D), jnp.float32),
            pltpu.SemaphoreType.DMA((2,)),
        ],
    )
    return pl.pallas_call(
        _mlp_body,
        grid_spec=grid_spec,
        out_shape=jax.ShapeDtypeStruct((NP, D), jnp.float32),
        compiler_params=pltpu.CompilerParams(
            vmem_limit_bytes=100 * 1024 * 1024),
    )(te, tv, first, slot, nval, nexp, x_sorted, gup, down, w_sorted)


_CHUNK = 16
_NCH = TPW // _CHUNK


def _combine_body(inv_hbm, y_hbm, out_hbm, ib, yb0, yb1, ob0, ob1, sem0, sem1, osem):
    wid = _wid()
    for c in range(_NCH):
        pltpu.sync_copy(
            inv_hbm.at[pl.ds(wid * K * TPW + c * K * _CHUNK, K * _CHUNK)],
            ib.at[c])
    ybs = (yb0, yb1)
    obs = (ob0, ob1)
    sems = (sem0, sem1)
    cps = [None, None]
    cps[0] = pltpu.async_copy(y_hbm.at[ib.at[0]], yb0, sem0)
    for c in range(_NCH):
        p = c % 2
        if c + 1 < _NCH:
            cps[1 - p] = pltpu.async_copy(
                y_hbm.at[ib.at[c + 1]], ybs[1 - p], sem=sems[1 - p])
        cps[p].wait()
        yb = ybs[p]
        ob = obs[p]
        if c >= 2:
            cps_out[p].wait()

        def body(j, carry):
            for col in range(D // 16):
                s = pl.ds(col * 16, 16)
                ob[j, s] = yb[2 * j, s] + yb[2 * j + 1, s]
            return carry

        lax.fori_loop(0, _CHUNK, body, 0)
        cp_o = pltpu.async_copy(
            ob, out_hbm.at[pl.ds(wid * TPW + c * _CHUNK, _CHUNK)], osem)
        if c == 0:
            cps_out = [cp_o, None]
        else:
            cps_out[p] = cp_o
    cps_out[0].wait()
    cps_out[1].wait()


_combine = pl.kernel(
    _combine_body,
    out_type=jax.ShapeDtypeStruct((T, D), jnp.float32),
    mesh=_MESH,
    compiler_params=pltpu.CompilerParams(needs_layout_passes=False),
    scratch_types=[
        pltpu.VMEM((_NCH, K * _CHUNK), jnp.int32),
        pltpu.VMEM((K * _CHUNK, D), jnp.float32),
        pltpu.VMEM((K * _CHUNK, D), jnp.float32),
        pltpu.VMEM((_CHUNK, D), jnp.float32),
        pltpu.VMEM((_CHUNK, D), jnp.float32),
        pltpu.SemaphoreType.DMA,
        pltpu.SemaphoreType.DMA,
        pltpu.SemaphoreType.DMA,
    ],
)


def kernel(hidden_states, router_weight, gate_up_weight, down_weight):
    wr_pad = jnp.zeros((128, D), jnp.float32).at[:E, :].set(router_weight.T)
    logits_t = _router_logits(hidden_states, wr_pad)
    eidx, wts, ranks, counts = _route(logits_t)
    x_sorted, w_sorted, inv, meta = _dispatch(
        counts, eidx, wts, ranks, hidden_states)
    tile_expert = meta[:NT, 0]
    tile_valid = meta[:NT, 1]
    y = _grouped_mlp(tile_expert, tile_valid, x_sorted,
                     gate_up_weight, down_weight, w_sorted)
    return _combine(inv, y)
